# baseline (speedup 1.0000x reference)
import functools

import jax
import jax.numpy as jnp
from jax import lax
from jax.experimental import pallas as pl
from jax.experimental.pallas import tpu as pltpu
from jax.experimental.pallas import tpu_sc as plsc

DIM = 64
SEQ = 200
BATCH = 4096
NBLK = 3907
W2_ROWS = NBLK * 128


def _t_body(x_ref, o_ref):
    x = x_ref[...]
    o_ref[:, 0:64] = x[:, 0:128].T
    o_ref[:, 64:128] = x[:, 128:256].T


@jax.jit
def _transpose_pack(wT):
    return pl.pallas_call(
        _t_body,
        grid=(NBLK,),
        in_specs=[pl.BlockSpec((64, 256), lambda c: (0, c))],
        out_specs=pl.BlockSpec((128, 128), lambda c: (c, 0)),
        out_shape=jax.ShapeDtypeStruct((W2_ROWS, 128), jnp.float32),
    )(wT)


def _gather_body(ids_hbm, w2_hbm, o5_hbm, ids_v, idx2_v, rows_v, out_v,
                 gsem, osem):
    t = lax.axis_index("s") * 2 + lax.axis_index("c")
    pltpu.sync_copy(ids_hbm.at[:, t], ids_v)

    iota = lax.iota(jnp.int32, 16)
    zero = jnp.zeros((16,), jnp.int32)

    def prep_and_fire(s, b):
        sr = s // 8
        si = s % 8
        for k in range(8):
            v_id = ids_v[sr, si, pl.ds(k * 16, 16)]
            idx2_v[b, pl.ds(k * 16, 16)] = (
                lax.shift_left(lax.shift_right_logical(v_id, 8), 7)
                + lax.bitwise_and(v_id, 127))
        pltpu.async_copy(w2_hbm.at[idx2_v.at[b]], rows_v.at[b], gsem.at[b])

    prep_and_fire(0, 0)

    @pl.loop(0, SEQ, step=2)
    def _s_step(s0):
        for b in range(2):
            s = s0 + b
            nxt = s + 1

            @pl.when(nxt < SEQ)
            def _fire():
                prep_and_fire(nxt, 1 - b)

            pltpu.make_async_copy(
                w2_hbm.at[idx2_v.at[b]], rows_v.at[b], gsem.at[b]).wait()

            @pl.when(s >= 2)
            def _drain():
                pltpu.make_async_copy(
                    out_v.at[b], o5_hbm.at[s - 2, :, t], osem.at[b]).wait()

            sr = s // 8
            si = s % 8

            @pl.loop(0, 8)
            def _k_grp(kk):
                v_id = ids_v[sr, si, pl.ds(kk * 16, 16)]
                hb = lax.shift_left(
                    lax.bitwise_and(lax.shift_right_logical(v_id, 7), 1), 6)
                jout = kk * 16 + iota
                jb = lax.shift_left(jout, 7) + hb

                @pl.loop(0, 16)
                def _diag(c0):
                    rot = lax.bitwise_and(iota + c0, 15)
                    jbrot = jb + rot
                    orot = jout + lax.shift_left(rot, 7)
                    for d4 in range(4):
                        v = plsc.load_gather(
                            rows_v.at[b], [zero, jbrot + d4 * 16])
                        plsc.store_scatter(
                            out_v.at[b], [zero, zero, orot + d4 * 2048], v)
            pltpu.async_copy(out_v.at[b], o5_hbm.at[s, :, t], osem.at[b])

    for b in range(2):
        pltpu.make_async_copy(
            out_v.at[b], o5_hbm.at[SEQ - 2 + b, :, t], osem.at[b]).wait()


@jax.jit
def _embedding_gather(ids5, w2):
    mesh = plsc.VectorSubcoreMesh(core_axis_name="c", subcore_axis_name="s")
    k = functools.partial(
        pl.kernel,
        mesh=mesh,
        out_type=jax.ShapeDtypeStruct((SEQ, 8, 32, 8, 128), jnp.float32),
        scratch_types=[
            pltpu.VMEM((25, 8, 128), jnp.int32),
            pltpu.VMEM((2, 128), jnp.int32),
            pltpu.VMEM((2, 128, 128), jnp.float32),
            pltpu.VMEM((2, 8, 8, 128), jnp.float32),
            pltpu.SemaphoreType.DMA((2,)),
            pltpu.SemaphoreType.DMA((2,)),
        ],
        compiler_params=pltpu.CompilerParams(
            use_tc_tiling_on_sc=False, needs_layout_passes=False),
    )(_gather_body)
    return k(ids5, w2)


def kernel(token_ids, weight):
    ids5 = token_ids.T.reshape(25, 8, 32, 128).transpose(0, 2, 1, 3)
    w2 = _transpose_pack(weight.T)
    o5 = _embedding_gather(ids5, w2)
    return o5.transpose(2, 4, 0, 1, 3).reshape(BATCH, SEQ, DIM)


# speedup vs baseline: 3.0723x; 3.0723x over previous
import functools

import jax
import jax.numpy as jnp
from jax import lax
from jax.experimental import pallas as pl
from jax.experimental.pallas import tpu as pltpu
from jax.experimental.pallas import tpu_sc as plsc

DIM = 64
SEQ = 200
BATCH = 4096
SB_PER_BLK = 8
NBLK = 489
W2_ROWS = NBLK * SB_PER_BLK * 128


def _t_body(x_ref, o_ref):
    x = x_ref[...]
    for u in range(SB_PER_BLK):
        o_ref[u * 128:(u + 1) * 128, 0:64] = x[:, u * 256:u * 256 + 128].T
        o_ref[u * 128:(u + 1) * 128, 64:128] = (
            x[:, u * 256 + 128:u * 256 + 256].T)


@jax.jit
def _transpose_pack(wT):
    return pl.pallas_call(
        _t_body,
        grid=(NBLK,),
        in_specs=[pl.BlockSpec((64, 256 * SB_PER_BLK), lambda c: (0, c))],
        out_specs=pl.BlockSpec((128 * SB_PER_BLK, 128), lambda c: (c, 0)),
        out_shape=jax.ShapeDtypeStruct((W2_ROWS, 128), jnp.float32),
    )(wT)


def _gather_body(ids_hbm, w2_hbm, o5_hbm, ids_v, idx2_v, rows_v, out_v,
                 gsem, osem):
    t = lax.axis_index("s") * 2 + lax.axis_index("c")
    pltpu.sync_copy(ids_hbm.at[:, t], ids_v)

    iota = lax.iota(jnp.int32, 16)
    zero = jnp.zeros((16,), jnp.int32)

    def prep_and_fire(s, b):
        sr = s // 8
        si = s % 8
        for k in range(8):
            v_id = ids_v[sr, si, pl.ds(k * 16, 16)]
            idx2_v[b, pl.ds(k * 16, 16)] = (
                lax.shift_left(lax.shift_right_logical(v_id, 8), 7)
                + lax.bitwise_and(v_id, 127))
        pltpu.async_copy(w2_hbm.at[idx2_v.at[b]], rows_v.at[b], gsem.at[b])

    prep_and_fire(0, 0)

    @pl.loop(0, SEQ, step=2)
    def _s_step(s0):
        for b in range(2):
            s = s0 + b
            nxt = s + 1

            @pl.when(nxt < SEQ)
            def _fire():
                prep_and_fire(nxt, 1 - b)

            pltpu.make_async_copy(
                w2_hbm.at[idx2_v.at[b]], rows_v.at[b], gsem.at[b]).wait()

            @pl.when(s >= 2)
            def _drain():
                pltpu.make_async_copy(
                    out_v.at[b], o5_hbm.at[s - 2, :, t], osem.at[b]).wait()

            sr = s // 8
            si = s % 8

            @pl.loop(0, 8)
            def _k_grp(kk):
                v_id = ids_v[sr, si, pl.ds(kk * 16, 16)]
                hb = lax.shift_left(
                    lax.bitwise_and(lax.shift_right_logical(v_id, 7), 1), 6)
                jout = kk * 16 + iota
                jb = lax.shift_left(jout, 7) + hb

                @pl.loop(0, 16, unroll=4)
                def _diag(c0):
                    rot = lax.bitwise_and(iota + c0, 15)
                    jbrot = jb + rot
                    orot = jout + lax.shift_left(rot, 7)
                    for d4 in range(4):
                        v = plsc.load_gather(
                            rows_v.at[b], [zero, jbrot + d4 * 16])
                        plsc.store_scatter(
                            out_v.at[b], [zero, zero, orot + d4 * 2048], v)
            pltpu.async_copy(out_v.at[b], o5_hbm.at[s, :, t], osem.at[b])

    for b in range(2):
        pltpu.make_async_copy(
            out_v.at[b], o5_hbm.at[SEQ - 2 + b, :, t], osem.at[b]).wait()


@jax.jit
def _embedding_gather(ids5, w2):
    mesh = plsc.VectorSubcoreMesh(core_axis_name="c", subcore_axis_name="s")
    k = functools.partial(
        pl.kernel,
        mesh=mesh,
        out_type=jax.ShapeDtypeStruct((SEQ, 8, 32, 8, 128), jnp.float32),
        scratch_types=[
            pltpu.VMEM((25, 8, 128), jnp.int32),
            pltpu.VMEM((2, 128), jnp.int32),
            pltpu.VMEM((2, 128, 128), jnp.float32),
            pltpu.VMEM((2, 8, 8, 128), jnp.float32),
            pltpu.SemaphoreType.DMA((2,)),
            pltpu.SemaphoreType.DMA((2,)),
        ],
        compiler_params=pltpu.CompilerParams(
            use_tc_tiling_on_sc=False, needs_layout_passes=False),
    )(_gather_body)
    return k(ids5, w2)


def kernel(token_ids, weight):
    ids5 = token_ids.T.reshape(25, 8, 32, 128).transpose(0, 2, 1, 3)
    w2 = _transpose_pack(weight.T)
    o5 = _embedding_gather(ids5, w2)
    return o5.transpose(2, 4, 0, 1, 3).reshape(BATCH, SEQ, DIM)


# speedup vs baseline: 3.9329x; 1.2801x over previous
import functools

import jax
import jax.numpy as jnp
from jax import lax
from jax.experimental import pallas as pl
from jax.experimental.pallas import tpu as pltpu
from jax.experimental.pallas import tpu_sc as plsc

DIM = 64
SEQ = 200
BATCH = 4096
SB_PER_BLK = 32
NBLK = 123
W2_ROWS = NBLK * SB_PER_BLK * 128


def _t_body(x_ref, o_ref):
    x = x_ref[...]
    for u in range(SB_PER_BLK):
        o_ref[u * 128:(u + 1) * 128, 0:64] = x[:, u * 256:u * 256 + 128].T
        o_ref[u * 128:(u + 1) * 128, 64:128] = (
            x[:, u * 256 + 128:u * 256 + 256].T)


@jax.jit
def _transpose_pack(wT):
    return pl.pallas_call(
        _t_body,
        grid=(NBLK,),
        in_specs=[pl.BlockSpec((64, 256 * SB_PER_BLK), lambda c: (0, c))],
        out_specs=pl.BlockSpec((128 * SB_PER_BLK, 128), lambda c: (c, 0)),
        out_shape=jax.ShapeDtypeStruct((W2_ROWS, 128), jnp.float32),
    )(wT)


def _gather_body(ids_hbm, w2_hbm, o5_hbm, ids_v, idx2_v, rows_v, out_v,
                 gsem, osem):
    t = lax.axis_index("s") * 2 + lax.axis_index("c")
    pltpu.sync_copy(ids_hbm.at[:, t], ids_v)

    iota = lax.iota(jnp.int32, 16)
    zero = jnp.zeros((16,), jnp.int32)

    def prep_and_fire(s, b):
        sr = s // 8
        si = s % 8
        for k in range(8):
            v_id = ids_v[sr, si, pl.ds(k * 16, 16)]
            idx2_v[b, pl.ds(k * 16, 16)] = (
                lax.shift_left(lax.shift_right_logical(v_id, 8), 7)
                + lax.bitwise_and(v_id, 127))
        pltpu.async_copy(w2_hbm.at[idx2_v.at[b]], rows_v.at[b], gsem.at[b])

    prep_and_fire(0, 0)

    @pl.loop(0, SEQ, step=2)
    def _s_step(s0):
        for b in range(2):
            s = s0 + b
            nxt = s + 1

            @pl.when(nxt < SEQ)
            def _fire():
                prep_and_fire(nxt, 1 - b)

            pltpu.make_async_copy(
                w2_hbm.at[idx2_v.at[b]], rows_v.at[b], gsem.at[b]).wait()

            @pl.when(s >= 2)
            def _drain():
                pltpu.make_async_copy(
                    out_v.at[b], o5_hbm.at[s - 2, :, t], osem.at[b]).wait()

            sr = s // 8
            si = s % 8

            @pl.loop(0, 8)
            def _k_grp(kk):
                v_id = ids_v[sr, si, pl.ds(kk * 16, 16)]
                hb = lax.shift_left(
                    lax.bitwise_and(lax.shift_right_logical(v_id, 7), 1), 6)
                jout = kk * 16 + iota
                jb = lax.shift_left(jout, 7) + hb

                @pl.loop(0, 16, unroll=4)
                def _diag(c0):
                    rot = lax.bitwise_and(iota + c0, 15)
                    jbrot = jb + rot
                    orot = jout + lax.shift_left(rot, 7)
                    for d4 in range(4):
                        v = plsc.load_gather(
                            rows_v.at[b], [zero, jbrot + d4 * 16])
                        plsc.store_scatter(
                            out_v.at[b], [zero, zero, orot + d4 * 2048], v)
            pltpu.async_copy(out_v.at[b], o5_hbm.at[s, :, t], osem.at[b])

    for b in range(2):
        pltpu.make_async_copy(
            out_v.at[b], o5_hbm.at[SEQ - 2 + b, :, t], osem.at[b]).wait()


@jax.jit
def _embedding_gather(ids5, w2):
    mesh = plsc.VectorSubcoreMesh(core_axis_name="c", subcore_axis_name="s")
    k = functools.partial(
        pl.kernel,
        mesh=mesh,
        out_type=jax.ShapeDtypeStruct((SEQ, 8, 32, 8, 128), jnp.float32),
        scratch_types=[
            pltpu.VMEM((25, 8, 128), jnp.int32),
            pltpu.VMEM((2, 128), jnp.int32),
            pltpu.VMEM((2, 128, 128), jnp.float32),
            pltpu.VMEM((2, 8, 8, 128), jnp.float32),
            pltpu.SemaphoreType.DMA((2,)),
            pltpu.SemaphoreType.DMA((2,)),
        ],
        compiler_params=pltpu.CompilerParams(
            use_tc_tiling_on_sc=False, needs_layout_passes=False),
    )(_gather_body)
    return k(ids5, w2)


def kernel(token_ids, weight):
    ids5 = token_ids.T.reshape(25, 8, 32, 128).transpose(0, 2, 1, 3)
    w2 = _transpose_pack(weight.T)
    o5 = _embedding_gather(ids5, w2)
    return o5.transpose(2, 4, 0, 1, 3).reshape(BATCH, SEQ, DIM)


# speedup vs baseline: 4.1121x; 1.0456x over previous
import functools

import jax
import jax.numpy as jnp
from jax import lax
from jax.experimental import pallas as pl
from jax.experimental.pallas import tpu as pltpu
from jax.experimental.pallas import tpu_sc as plsc

DIM = 64
SEQ = 200
BATCH = 4096
SB_PER_BLK = 64
NBLK = 62
W2_ROWS = NBLK * SB_PER_BLK * 128


def _t_body(x_ref, o_ref):
    x = x_ref[...]
    for u in range(SB_PER_BLK):
        o_ref[u * 128:(u + 1) * 128, 0:64] = x[:, u * 256:u * 256 + 128].T
        o_ref[u * 128:(u + 1) * 128, 64:128] = (
            x[:, u * 256 + 128:u * 256 + 256].T)


@jax.jit
def _transpose_pack(wT):
    return pl.pallas_call(
        _t_body,
        grid=(NBLK,),
        in_specs=[pl.BlockSpec((64, 256 * SB_PER_BLK), lambda c: (0, c))],
        out_specs=pl.BlockSpec((128 * SB_PER_BLK, 128), lambda c: (c, 0)),
        out_shape=jax.ShapeDtypeStruct((W2_ROWS, 128), jnp.float32),
    )(wT)


def _gather_body(ids_hbm, w2_hbm, o5_hbm, ids_v, idx2_v, rows_v, out_v,
                 gsem, osem):
    t = lax.axis_index("s") * 2 + lax.axis_index("c")
    pltpu.sync_copy(ids_hbm.at[:, t], ids_v)

    iota = lax.iota(jnp.int32, 16)
    zero = jnp.zeros((16,), jnp.int32)

    def prep_and_fire(s, b):
        sr = s // 8
        si = s % 8
        for k in range(8):
            v_id = ids_v[sr, si, pl.ds(k * 16, 16)]
            idx2_v[b, pl.ds(k * 16, 16)] = (
                lax.shift_left(lax.shift_right_logical(v_id, 8), 7)
                + lax.bitwise_and(v_id, 127))
        pltpu.async_copy(w2_hbm.at[idx2_v.at[b]], rows_v.at[b], gsem.at[b])

    prep_and_fire(0, 0)

    @pl.loop(0, SEQ, step=2)
    def _s_step(s0):
        for b in range(2):
            s = s0 + b
            nxt = s + 1

            @pl.when(nxt < SEQ)
            def _fire():
                prep_and_fire(nxt, 1 - b)

            pltpu.make_async_copy(
                w2_hbm.at[idx2_v.at[b]], rows_v.at[b], gsem.at[b]).wait()

            @pl.when(s >= 2)
            def _drain():
                pltpu.make_async_copy(
                    out_v.at[b], o5_hbm.at[s - 2, :, t], osem.at[b]).wait()

            sr = s // 8
            si = s % 8

            @pl.loop(0, 8)
            def _k_grp(kk):
                v_id = ids_v[sr, si, pl.ds(kk * 16, 16)]
                hb = lax.shift_left(
                    lax.bitwise_and(lax.shift_right_logical(v_id, 7), 1), 6)
                jout = kk * 16 + iota
                jb = lax.shift_left(jout, 7) + hb

                @pl.loop(0, 16, unroll=4)
                def _diag(c0):
                    rot = lax.bitwise_and(iota + c0, 15)
                    jbrot = jb + rot
                    orot = jout + lax.shift_left(rot, 7)
                    for d4 in range(4):
                        v = plsc.load_gather(
                            rows_v.at[b], [zero, jbrot + d4 * 16])
                        plsc.store_scatter(
                            out_v.at[b], [zero, zero, orot + d4 * 2048], v)
            pltpu.async_copy(out_v.at[b], o5_hbm.at[s, :, t], osem.at[b])

    for b in range(2):
        pltpu.make_async_copy(
            out_v.at[b], o5_hbm.at[SEQ - 2 + b, :, t], osem.at[b]).wait()


@jax.jit
def _embedding_gather(ids5, w2):
    mesh = plsc.VectorSubcoreMesh(core_axis_name="c", subcore_axis_name="s")
    k = functools.partial(
        pl.kernel,
        mesh=mesh,
        out_type=jax.ShapeDtypeStruct((SEQ, 8, 32, 8, 128), jnp.float32),
        scratch_types=[
            pltpu.VMEM((25, 8, 128), jnp.int32),
            pltpu.VMEM((2, 128), jnp.int32),
            pltpu.VMEM((2, 128, 128), jnp.float32),
            pltpu.VMEM((2, 8, 8, 128), jnp.float32),
            pltpu.SemaphoreType.DMA((2,)),
            pltpu.SemaphoreType.DMA((2,)),
        ],
        compiler_params=pltpu.CompilerParams(
            use_tc_tiling_on_sc=False, needs_layout_passes=False),
    )(_gather_body)
    return k(ids5, w2)


def kernel(token_ids, weight):
    ids5 = token_ids.T.reshape(25, 8, 32, 128).transpose(0, 2, 1, 3)
    w2 = _transpose_pack(weight.T)
    o5 = _embedding_gather(ids5, w2)
    return o5.transpose(2, 4, 0, 1, 3).reshape(BATCH, SEQ, DIM)
